# Initial kernel scaffold; baseline (speedup 1.0000x reference)
import jax
import jax.numpy as jnp
from jax.experimental import pallas as pl


def kernel(edge_vals, user_emb, item_emb, users, items, negatives, edge_index):
    raise NotImplementedError("write your pallas kernel here")


# speedup vs baseline: 2.1779x; 2.1779x over previous
import jax
import jax.numpy as jnp
from jax import lax
from jax.experimental import pallas as pl
from jax.experimental.pallas import tpu as pltpu
from jax.experimental.pallas import tpu_sc as plsc

N_USERS = 2000
N_ITEMS = 8000
NN = N_USERS + N_ITEMS
D = 128
HOPS = 3
NNEG = 4
BATCH = 1024
E = 320000

NC = 2
NS = 16
HALF = NN // NC
ACC_ROWS = 5008
EC = 80
CHUNKS = 250
EPW = CHUNKS * EC

XROWS_PER_SUB = 624
XTAIL = NN - NS * XROWS_PER_SUB


def _hop_body(x_hbm, row_hbm, col_hbm, val_hbm, zero_hbm, out_hbm,
              x_sh, acc_sh, rowv, colv, lrowv, vals_sm, rows, sem):
    c = lax.axis_index("c")
    s = lax.axis_index("s")
    lo = c * HALF
    trash = HALF + (s % 8)

    pltpu.sync_copy(x_hbm.at[pl.ds(s * XROWS_PER_SUB, XROWS_PER_SUB)],
                    x_sh.at[pl.ds(s * XROWS_PER_SUB, XROWS_PER_SUB)])

    @pl.when(s == 0)
    def _():
        pltpu.sync_copy(x_hbm.at[pl.ds(NS * XROWS_PER_SUB, XTAIL)],
                        x_sh.at[pl.ds(NS * XROWS_PER_SUB, XTAIL)])

    @pl.when(s == 1)
    def _():
        pltpu.sync_copy(zero_hbm, acc_sh)
    plsc.subcore_barrier()

    base = s * EPW

    def chunk(j, carry):
        off = base + j * EC
        pltpu.sync_copy(row_hbm.at[pl.ds(off, EC)], rowv)
        pltpu.sync_copy(col_hbm.at[pl.ds(off, EC)], colv)
        pltpu.sync_copy(val_hbm.at[pl.ds(off, EC)], vals_sm)
        for q in range(EC // 16):
            r = rowv[pl.ds(q * 16, 16)]
            lr = r - lo
            ok = (lr >= 0) & (lr < HALF)
            lrowv[pl.ds(q * 16, 16)] = jnp.where(ok, lr, trash)
        pltpu.async_copy(x_sh.at[colv], rows, sem).wait()

        def scale16(g, _):
            vv = vals_sm[pl.ds(g * 16, 16)]
            for l in range(16):
                e = g * 16 + l
                v = vv[l]
                for q in range(D // 16):
                    rows[e, pl.ds(q * 16, 16)] = rows[e, pl.ds(q * 16, 16)] * v
            return 0

        lax.fori_loop(0, EC // 16, scale16, 0)
        pltpu.sync_copy(rows, acc_sh.at[lrowv], add=True)
        return carry

    lax.fori_loop(0, CHUNKS, chunk, 0)
    plsc.subcore_barrier()

    @pl.when(s == 0)
    def _():
        pltpu.sync_copy(acc_sh.at[pl.ds(0, HALF)], out_hbm.at[pl.ds(lo, HALF)])


_hop = pl.kernel(
    _hop_body,
    out_type=jax.ShapeDtypeStruct((NN, D), jnp.float32),
    mesh=plsc.VectorSubcoreMesh(core_axis_name="c", subcore_axis_name="s"),
    scratch_types=[
        pltpu.VMEM_SHARED((NN, D), jnp.float32),
        pltpu.VMEM_SHARED((ACC_ROWS, D), jnp.float32),
        pltpu.VMEM((EC,), jnp.int32),
        pltpu.VMEM((EC,), jnp.int32),
        pltpu.VMEM((EC,), jnp.int32),
        pltpu.VMEM((EC,), jnp.float32),
        pltpu.VMEM((EC, D), jnp.float32),
        pltpu.SemaphoreType.DMA,
    ],
)

BPW = BATCH // (NC * NS)


def _loss_body(x0, x1, x2, x3, u_hbm, i_hbm, n_hbm, pos_out, neg_out,
               idxv, tmp, usum, isum, ng0, ng1, ng2, ng3, pos_sm, neg_sm, sem):
    c = lax.axis_index("c")
    s = lax.axis_index("s")
    w = s * NC + c
    b0 = w * BPW
    tables = (x0, x1, x2, x3)
    ngs = (ng0, ng1, ng2, ng3)

    def gather_sum(idx_hbm, off, dst):
        pltpu.sync_copy(idx_hbm.at[pl.ds(off, BPW)], idxv)
        pltpu.async_copy(tables[0].at[idxv], dst, sem).wait()
        for t in range(1, 4):
            pltpu.async_copy(tables[t].at[idxv], tmp, sem).wait()

            def addloop(i, _):
                for q in range(D // 16):
                    dst[i, pl.ds(q * 16, 16)] = (dst[i, pl.ds(q * 16, 16)]
                                                 + tmp[i, pl.ds(q * 16, 16)])
                return 0

            lax.fori_loop(0, BPW, addloop, 0)

    gather_sum(u_hbm, b0, usum)
    gather_sum(i_hbm, b0, isum)
    for n in range(NNEG):
        gather_sum(n_hbm, n * BATCH + b0, ngs[n])

    def dots(b, _):
        pacc = jnp.zeros((16,), jnp.float32)
        for q in range(D // 16):
            pacc = pacc + (usum[b, pl.ds(q * 16, 16)]
                           * isum[b, pl.ds(q * 16, 16)])
        pos_sm[b, pl.ds(0, 16)] = pacc
        for n in range(NNEG):
            nacc = jnp.zeros((16,), jnp.float32)
            for q in range(D // 16):
                nacc = nacc + (usum[b, pl.ds(q * 16, 16)]
                               * ngs[n][b, pl.ds(q * 16, 16)])
            neg_sm[n * BPW + b, pl.ds(0, 16)] = nacc
        return 0

    lax.fori_loop(0, BPW, dots, 0)
    pltpu.sync_copy(pos_sm, pos_out.at[pl.ds(b0, BPW)])
    for n in range(NNEG):
        pltpu.sync_copy(neg_sm.at[pl.ds(n * BPW, BPW)],
                        neg_out.at[pl.ds(n * BATCH + b0, BPW)])


_loss = pl.kernel(
    _loss_body,
    out_type=(jax.ShapeDtypeStruct((BATCH, 16), jnp.float32),
              jax.ShapeDtypeStruct((NNEG * BATCH, 16), jnp.float32)),
    mesh=plsc.VectorSubcoreMesh(core_axis_name="c", subcore_axis_name="s"),
    scratch_types=[
        pltpu.VMEM((BPW,), jnp.int32),
        pltpu.VMEM((BPW, D), jnp.float32),
        pltpu.VMEM((BPW, D), jnp.float32),
        pltpu.VMEM((BPW, D), jnp.float32),
        pltpu.VMEM((BPW, D), jnp.float32),
        pltpu.VMEM((BPW, D), jnp.float32),
        pltpu.VMEM((BPW, D), jnp.float32),
        pltpu.VMEM((BPW, D), jnp.float32),
        pltpu.VMEM((BPW, 16), jnp.float32),
        pltpu.VMEM((NNEG * BPW, 16), jnp.float32),
        pltpu.SemaphoreType.DMA,
    ],
)


def _nce_body(p_ref, n_ref, o_ref):
    p = jnp.sum(p_ref[...], axis=-1) * (1.0 / 16.0)
    nk = jnp.sum(n_ref[...], axis=-1) * (1.0 / 16.0)
    ne = jnp.sum(jnp.exp(nk), axis=0)
    loss = jnp.mean(jnp.log(jnp.exp(p) + ne) - p)
    o_ref[...] = jnp.reshape(loss, (1, 1))


_nce = pl.pallas_call(
    _nce_body,
    out_shape=jax.ShapeDtypeStruct((1, 1), jnp.float32),
)


def kernel(edge_vals, user_emb, item_emb, users, items, negatives, edge_index):
    all_emb = jnp.concatenate([user_emb, item_emb], axis=0).astype(jnp.float32)
    rowp = edge_index[0].astype(jnp.int32)
    colp = edge_index[1].astype(jnp.int32)
    evp = edge_vals.astype(jnp.float32)
    zero_acc = jnp.zeros((ACC_ROWS, D), jnp.float32)

    x0 = all_emb
    x1 = _hop(x0, rowp, colp, evp, zero_acc)
    x2 = _hop(x1, rowp, colp, evp, zero_acc)
    x3 = _hop(x2, rowp, colp, evp, zero_acc)

    u = users.astype(jnp.int32)
    it = items.astype(jnp.int32) + N_USERS
    ng = negatives.astype(jnp.int32) + N_USERS
    pos, negk = _loss(x0, x1, x2, x3, u, it, ng)
    out = _nce(pos, negk.reshape(NNEG, BATCH, 16))
    return out[0, 0]


# speedup vs baseline: 4.5531x; 2.0906x over previous
import jax
import jax.numpy as jnp
from jax import lax
from jax.experimental import pallas as pl
from jax.experimental.pallas import tpu as pltpu
from jax.experimental.pallas import tpu_sc as plsc

N_USERS = 2000
N_ITEMS = 8000
NN = N_USERS + N_ITEMS
D = 128
HOPS = 3
NNEG = 4
BATCH = 1024
E = 320000

NC = 2
NS = 16
HALF = NN // NC
ACC_ROWS = 5024
EC = 32
CHUNKS = 625
EPW = CHUNKS * EC
PAIRS = CHUNKS // 2
EPAD = EPW * NS + EC

XROWS_PER_SUB = 624
XTAIL = NN - NS * XROWS_PER_SUB


def _hop_body(x_hbm, row_hbm, col_hbm, val_hbm, zero_hbm, out_hbm,
              x_sh, acc_sh,
              rowA, colA, valA, lrowA, rowsA,
              rowB, colB, valB, lrowB, rowsB,
              semMA, semMB, semGA, semGB, semSA, semSB):
    c = lax.axis_index("c")
    s = lax.axis_index("s")
    lo = c * HALF
    trash = HALF + s

    pltpu.sync_copy(x_hbm.at[pl.ds(s * XROWS_PER_SUB, XROWS_PER_SUB)],
                    x_sh.at[pl.ds(s * XROWS_PER_SUB, XROWS_PER_SUB)])

    @pl.when(s == 0)
    def _():
        pltpu.sync_copy(x_hbm.at[pl.ds(NS * XROWS_PER_SUB, XTAIL)],
                        x_sh.at[pl.ds(NS * XROWS_PER_SUB, XTAIL)])

    @pl.when(s == 1)
    def _():
        pltpu.sync_copy(zero_hbm, acc_sh)
    plsc.subcore_barrier()

    base = s * EPW
    A = (rowA, colA, valA, lrowA, rowsA, semMA, semGA, semSA)
    Bb = (rowB, colB, valB, lrowB, rowsB, semMB, semGB, semSB)

    def meta_issue(j, bufs):
        rowb, colb, valb = bufs[0], bufs[1], bufs[2]
        semM = bufs[5]
        off = base + j * EC
        pltpu.async_copy(row_hbm.at[pl.ds(off, EC)], rowb, semM)
        pltpu.async_copy(col_hbm.at[pl.ds(off, EC)], colb, semM)
        pltpu.async_copy(val_hbm.at[pl.ds(off, EC)], valb, semM)

    def meta_drain(j, bufs):
        rowb, colb, valb = bufs[0], bufs[1], bufs[2]
        semM = bufs[5]
        off = base + j * EC
        pltpu.make_async_copy(row_hbm.at[pl.ds(off, EC)], rowb, semM).wait()
        pltpu.make_async_copy(col_hbm.at[pl.ds(off, EC)], colb, semM).wait()
        pltpu.make_async_copy(val_hbm.at[pl.ds(off, EC)], valb, semM).wait()

    def start_phase(j, bufs):
        rowb, colb, lrowb, rowsb = bufs[0], bufs[1], bufs[3], bufs[4]
        semG, semS = bufs[6], bufs[7]
        meta_drain(j, bufs)
        for q in range(EC // 16):
            r = rowb[pl.ds(q * 16, 16)]
            lr = r - lo
            ok = (lr >= 0) & (lr < HALF)
            lrowb[pl.ds(q * 16, 16)] = jnp.where(ok, lr, trash)
        pltpu.make_async_copy(rowsb, acc_sh.at[lrowb], semS).wait()
        pltpu.async_copy(x_sh.at[colb], rowsb, semG)

    def finish_phase(bufs, next_meta_j):
        colb, valb, lrowb, rowsb = bufs[1], bufs[2], bufs[3], bufs[4]
        semG, semS = bufs[6], bufs[7]
        pltpu.make_async_copy(x_sh.at[colb], rowsb, semG).wait()

        def scale16(g, _):
            vv = valb[pl.ds(g * 16, 16)]
            for l in range(16):
                e = g * 16 + l
                v = vv[l]
                for q in range(D // 16):
                    rowsb[e, pl.ds(q * 16, 16)] = rowsb[e, pl.ds(q * 16, 16)] * v
            return 0

        lax.fori_loop(0, EC // 16, scale16, 0)
        pltpu.async_copy(rowsb, acc_sh.at[lrowb], semS)
        if next_meta_j is not None:
            meta_issue(next_meta_j, bufs)

    for q in range(EC // 16):
        tr = jnp.broadcast_to(trash, (16,)).astype(jnp.int32)
        lrowA[pl.ds(q * 16, 16)] = tr
        lrowB[pl.ds(q * 16, 16)] = tr
    pltpu.async_copy(rowsA, acc_sh.at[lrowA], semSA)
    pltpu.async_copy(rowsB, acc_sh.at[lrowB], semSB)
    meta_issue(0, A)
    meta_issue(1, Bb)

    def pair(k, carry):
        jA = 2 * k
        jB = 2 * k + 1
        start_phase(jA, A)
        start_phase(jB, Bb)
        finish_phase(A, jA + 2)
        finish_phase(Bb, jB + 2)
        return carry

    lax.fori_loop(0, PAIRS, pair, 0)
    start_phase(CHUNKS - 1, A)
    finish_phase(A, None)
    pltpu.make_async_copy(rowsA, acc_sh.at[lrowA], semSA).wait()
    pltpu.make_async_copy(rowsB, acc_sh.at[lrowB], semSB).wait()
    meta_drain(CHUNKS, Bb)
    plsc.subcore_barrier()

    @pl.when(s == 0)
    def _():
        pltpu.sync_copy(acc_sh.at[pl.ds(0, HALF)], out_hbm.at[pl.ds(lo, HALF)])


_hop = pl.kernel(
    _hop_body,
    out_type=jax.ShapeDtypeStruct((NN, D), jnp.float32),
    mesh=plsc.VectorSubcoreMesh(core_axis_name="c", subcore_axis_name="s"),
    scratch_types=[
        pltpu.VMEM_SHARED((NN, D), jnp.float32),
        pltpu.VMEM_SHARED((ACC_ROWS, D), jnp.float32),
        pltpu.VMEM((EC,), jnp.int32),
        pltpu.VMEM((EC,), jnp.int32),
        pltpu.VMEM((EC,), jnp.float32),
        pltpu.VMEM((EC,), jnp.int32),
        pltpu.VMEM((EC, D), jnp.float32),
        pltpu.VMEM((EC,), jnp.int32),
        pltpu.VMEM((EC,), jnp.int32),
        pltpu.VMEM((EC,), jnp.float32),
        pltpu.VMEM((EC,), jnp.int32),
        pltpu.VMEM((EC, D), jnp.float32),
        pltpu.SemaphoreType.DMA,
        pltpu.SemaphoreType.DMA,
        pltpu.SemaphoreType.DMA,
        pltpu.SemaphoreType.DMA,
        pltpu.SemaphoreType.DMA,
        pltpu.SemaphoreType.DMA,
    ],
)

BPW = BATCH // (NC * NS)


def _loss_body(x0, x1, x2, x3, u_hbm, i_hbm, n_hbm, pos_out, neg_out,
               idxv, tmp, usum, isum, ng0, ng1, ng2, ng3, pos_sm, neg_sm, sem):
    c = lax.axis_index("c")
    s = lax.axis_index("s")
    w = s * NC + c
    b0 = w * BPW
    tables = (x0, x1, x2, x3)
    ngs = (ng0, ng1, ng2, ng3)

    def gather_sum(idx_hbm, off, dst):
        pltpu.sync_copy(idx_hbm.at[pl.ds(off, BPW)], idxv)
        pltpu.async_copy(tables[0].at[idxv], dst, sem).wait()
        for t in range(1, 4):
            pltpu.async_copy(tables[t].at[idxv], tmp, sem).wait()

            def addloop(i, _):
                for q in range(D // 16):
                    dst[i, pl.ds(q * 16, 16)] = (dst[i, pl.ds(q * 16, 16)]
                                                 + tmp[i, pl.ds(q * 16, 16)])
                return 0

            lax.fori_loop(0, BPW, addloop, 0)

    gather_sum(u_hbm, b0, usum)
    gather_sum(i_hbm, b0, isum)
    for n in range(NNEG):
        gather_sum(n_hbm, n * BATCH + b0, ngs[n])

    def dots(b, _):
        pacc = jnp.zeros((16,), jnp.float32)
        for q in range(D // 16):
            pacc = pacc + (usum[b, pl.ds(q * 16, 16)]
                           * isum[b, pl.ds(q * 16, 16)])
        pos_sm[b, pl.ds(0, 16)] = pacc
        for n in range(NNEG):
            nacc = jnp.zeros((16,), jnp.float32)
            for q in range(D // 16):
                nacc = nacc + (usum[b, pl.ds(q * 16, 16)]
                               * ngs[n][b, pl.ds(q * 16, 16)])
            neg_sm[n * BPW + b, pl.ds(0, 16)] = nacc
        return 0

    lax.fori_loop(0, BPW, dots, 0)
    pltpu.sync_copy(pos_sm, pos_out.at[pl.ds(b0, BPW)])
    for n in range(NNEG):
        pltpu.sync_copy(neg_sm.at[pl.ds(n * BPW, BPW)],
                        neg_out.at[pl.ds(n * BATCH + b0, BPW)])


_loss = pl.kernel(
    _loss_body,
    out_type=(jax.ShapeDtypeStruct((BATCH, 16), jnp.float32),
              jax.ShapeDtypeStruct((NNEG * BATCH, 16), jnp.float32)),
    mesh=plsc.VectorSubcoreMesh(core_axis_name="c", subcore_axis_name="s"),
    scratch_types=[
        pltpu.VMEM((BPW,), jnp.int32),
        pltpu.VMEM((BPW, D), jnp.float32),
        pltpu.VMEM((BPW, D), jnp.float32),
        pltpu.VMEM((BPW, D), jnp.float32),
        pltpu.VMEM((BPW, D), jnp.float32),
        pltpu.VMEM((BPW, D), jnp.float32),
        pltpu.VMEM((BPW, D), jnp.float32),
        pltpu.VMEM((BPW, D), jnp.float32),
        pltpu.VMEM((BPW, 16), jnp.float32),
        pltpu.VMEM((NNEG * BPW, 16), jnp.float32),
        pltpu.SemaphoreType.DMA,
    ],
)


def _nce_body(p_ref, n_ref, o_ref):
    p = jnp.sum(p_ref[...], axis=-1) * (1.0 / 16.0)
    nk = jnp.sum(n_ref[...], axis=-1) * (1.0 / 16.0)
    ne = jnp.sum(jnp.exp(nk), axis=0)
    loss = jnp.mean(jnp.log(jnp.exp(p) + ne) - p)
    o_ref[...] = jnp.reshape(loss, (1, 1))


_nce = pl.pallas_call(
    _nce_body,
    out_shape=jax.ShapeDtypeStruct((1, 1), jnp.float32),
)


def kernel(edge_vals, user_emb, item_emb, users, items, negatives, edge_index):
    all_emb = jnp.concatenate([user_emb, item_emb], axis=0).astype(jnp.float32)
    pad = EPAD - E
    rowp = jnp.concatenate([edge_index[0].astype(jnp.int32),
                            jnp.full((pad,), NN, jnp.int32)])
    colp = jnp.concatenate([edge_index[1].astype(jnp.int32),
                            jnp.zeros((pad,), jnp.int32)])
    evp = jnp.concatenate([edge_vals.astype(jnp.float32),
                           jnp.zeros((pad,), jnp.float32)])
    zero_acc = jnp.zeros((ACC_ROWS, D), jnp.float32)

    x0 = all_emb
    x1 = _hop(x0, rowp, colp, evp, zero_acc)
    x2 = _hop(x1, rowp, colp, evp, zero_acc)
    x3 = _hop(x2, rowp, colp, evp, zero_acc)

    u = users.astype(jnp.int32)
    it = items.astype(jnp.int32) + N_USERS
    ng = negatives.astype(jnp.int32) + N_USERS
    pos, negk = _loss(x0, x1, x2, x3, u, it, ng)
    out = _nce(pos, negk.reshape(NNEG, BATCH, 16))
    return out[0, 0]
